# baseline (speedup 1.0000x reference)
import jax
import jax.numpy as jnp
from jax import lax
from jax.experimental import pallas as pl
from jax.experimental.pallas import tpu as pltpu
from jax.experimental.pallas import tpu_sc as plsc

N_NODES = 10000
N_EDGES = 320000
D_IN = 128
N_HEADS = 8
UNITS = 16
HU = N_HEADS * UNITS
AW = HU + 16

CHUNK = 100
N_CHUNKS = N_EDGES // CHUNK
NW = 32
NBASE = N_CHUNKS // NW
NQUAD = NBASE // 4
TGO = 112
IW = 2 * TGO
N_GROUPS = N_NODES // 16


def _k1_body(x_ref, w_ref, at_ref, as_ref, xpa_ref, ftt_ref, mf_ref):
    i = pl.program_id(0)
    xb = x_ref[...]
    xp = jnp.dot(xb, w_ref[...], preferred_element_type=jnp.float32,
                 precision=lax.Precision.HIGHEST)
    ftt_ref[...] = jnp.dot(xp, at_ref[...], preferred_element_type=jnp.float32,
                           precision=lax.Precision.HIGHEST)
    fss = jnp.dot(xp, as_ref[...], preferred_element_type=jnp.float32,
                  precision=lax.Precision.HIGHEST)
    xpa_ref[...] = jnp.concatenate([xp, fss], axis=1)
    bm = jnp.max(fss, axis=0, keepdims=True)

    @pl.when(i == 0)
    def _():
        mf_ref[...] = bm

    @pl.when(i > 0)
    def _():
        mf_ref[...] = jnp.maximum(mf_ref[...], bm)


def _k1(x, w, a_t, a_s):
    blk = 1000
    grid = N_NODES // blk
    return pl.pallas_call(
        _k1_body,
        grid=(grid,),
        in_specs=[
            pl.BlockSpec((blk, D_IN), lambda i: (i, 0)),
            pl.BlockSpec((D_IN, HU), lambda i: (0, 0)),
            pl.BlockSpec((D_IN, 16), lambda i: (0, 0)),
            pl.BlockSpec((D_IN, 16), lambda i: (0, 0)),
        ],
        out_specs=[
            pl.BlockSpec((blk, AW), lambda i: (i, 0)),
            pl.BlockSpec((blk, 16), lambda i: (i, 0)),
            pl.BlockSpec((1, 16), lambda i: (0, 0)),
        ],
        out_shape=[
            jax.ShapeDtypeStruct((N_NODES, AW), jnp.float32),
            jax.ShapeDtypeStruct((N_NODES, 16), jnp.float32),
            jax.ShapeDtypeStruct((1, 16), jnp.float32),
        ],
    )(x, w, a_t, a_s)


def _k2_body(stc_h, ftt_h, xpa_h, mfs_h,
             acc_o,
             ia0, ia1, ib0, ib1, xpbufa, xpbufb, tbufa, tbufb, tixa, tixb,
             mfs_v, nacc,
             sga, sgb, ssa, ssb, sia0, sia1, sib0, sib1):
    cid = lax.axis_index("c")
    sid = lax.axis_index("s")
    wid = sid * 2 + cid

    zer = jnp.zeros((16,), jnp.float32)

    def zb(k, c):
        xpbufa[k // 9, pl.ds((k % 9) * 16, 16)] = zer
        return c
    lax.fori_loop(0, 16 * 9, zb, 0)

    n_my_g = (N_GROUPS - sid + 15) // 16

    def zg(k, c):
        r0 = (sid + k * 16) * 16
        pltpu.sync_copy(xpbufa.at[pl.ds(0, 16)], nacc.at[pl.ds(r0, 16)])
        return c
    lax.fori_loop(0, n_my_g, zg, 0)
    plsc.subcore_barrier()

    pltpu.sync_copy(mfs_h, mfs_v)
    mfs = mfs_v[:]
    hidx = [jnp.full((16,), h, jnp.int32) for h in range(N_HEADS)]

    gdn = lax.GatherDimensionNumbers(
        offset_dims=(), collapsed_slice_dims=(0,), start_index_map=(0,))

    def take16(vec, idx):
        return lax.gather(
            vec, idx[:, None], gdn, (1,),
            mode=lax.GatherScatterMode.PROMISE_IN_BOUNDS)

    seta = (xpbufa, tbufa, tixa, sga, ssa)
    setb = (xpbufb, tbufb, tixb, sgb, ssb)

    def idx_dma(c, ibuf, isem):
        return pltpu.make_async_copy(stc_h.at[wid + c * NW], ibuf, isem)

    def g_descs(bufs, ibuf):
        xpbuf, tbuf, _, sg, _ = bufs
        return (
            pltpu.make_async_copy(xpa_h.at[ibuf.at[pl.ds(0, CHUNK)]],
                                  xpbuf, sg),
            pltpu.make_async_copy(ftt_h.at[ibuf.at[pl.ds(TGO, CHUNK)]],
                                  tbuf, sg),
        )

    def s_desc(bufs):
        xpbuf, _, tix, _, ss = bufs
        return pltpu.make_async_copy(xpbuf, nacc.at[tix], ss)

    def prefetch(bufs, ibuf, isem, first=False):
        _, _, tix, _, _ = bufs
        if not first:
            s_desc(bufs).wait()
        idx_dma(0, ibuf, isem).wait()
        for k in range(CHUNK // 16):
            tix[pl.ds(k * 16, 16)] = ibuf[pl.ds(TGO + k * 16, 16)]
        tix[pl.ds(CHUNK - 16, 16)] = ibuf[pl.ds(TGO + CHUNK - 16, 16)]
        for d in g_descs(bufs, ibuf):
            d.start()

    def process(bufs, ibuf, refill_c, isem):
        xpbuf, tbuf, tix, sg, ss = bufs
        for d in g_descs(bufs, ibuf):
            d.wait()

        @pl.when(refill_c < NBASE)
        def _():
            idx_dma(refill_c, ibuf, isem).start()

        @plsc.parallel_loop(0, CHUNK, unroll=4)
        def _(e):
            rt = tbuf[e, :]
            rs = xpbuf[e, pl.ds(HU, 16)]
            s = rt + rs
            s = jnp.maximum(s, 0.2 * s)
            cm = rt + mfs
            cm = jnp.maximum(cm, 0.2 * cm)
            p = jnp.exp(s - cm)
            xpbuf[e, pl.ds(HU, 16)] = p
            for h in range(N_HEADS):
                wv = take16(p, hidx[h])
                xv = xpbuf[e, pl.ds(h * UNITS, UNITS)]
                xpbuf[e, pl.ds(h * UNITS, UNITS)] = xv * wv

        pltpu.async_copy(xpbuf, nacc.at[tix], ss, add=True)

    idx_dma(0, ia0, sia0).start()
    idx_dma(1, ib0, sib0).start()
    idx_dma(2, ia1, sia1).start()
    idx_dma(3, ib1, sib1).start()
    prefetch(seta, ia0, sia0, first=True)
    prefetch(setb, ib0, sib0, first=True)

    def quad_body(k, cc):
        c = 4 * k
        process(seta, ia0, c + 4, sia0)
        prefetch(seta, ia1, sia1)
        process(setb, ib0, c + 5, sib0)
        prefetch(setb, ib1, sib1)
        process(seta, ia1, c + 6, sia1)

        @pl.when(c + 4 < NBASE)
        def _():
            prefetch(seta, ia0, sia0)
        process(setb, ib1, c + 7, sib1)

        @pl.when(c + 5 < NBASE)
        def _():
            prefetch(setb, ib0, sib0)
        return cc
    lax.fori_loop(0, NQUAD, quad_body, 0)

    s_desc(seta).wait()
    s_desc(setb).wait()

    plsc.subcore_barrier()

    def dg(k, c):
        r0 = (sid + k * 16) * 16
        pltpu.sync_copy(nacc.at[pl.ds(r0, 16)], acc_o.at[cid, pl.ds(r0, 16)])
        return c
    lax.fori_loop(0, n_my_g, dg, 0)


def _k2(stc, ftt, xpa, mfs):
    mesh = plsc.VectorSubcoreMesh(core_axis_name="c", subcore_axis_name="s")
    f = pl.kernel(
        _k2_body,
        mesh=mesh,
        out_type=jax.ShapeDtypeStruct((2, N_NODES, AW), jnp.float32),
        scratch_types=[
            pltpu.VMEM((IW,), jnp.int32),
            pltpu.VMEM((IW,), jnp.int32),
            pltpu.VMEM((IW,), jnp.int32),
            pltpu.VMEM((IW,), jnp.int32),
            pltpu.VMEM((CHUNK, AW), jnp.float32),
            pltpu.VMEM((CHUNK, AW), jnp.float32),
            pltpu.VMEM((CHUNK, 16), jnp.float32),
            pltpu.VMEM((CHUNK, 16), jnp.float32),
            pltpu.VMEM((CHUNK,), jnp.int32),
            pltpu.VMEM((CHUNK,), jnp.int32),
            pltpu.VMEM((16,), jnp.float32),
            pltpu.VMEM_SHARED((N_NODES, AW), jnp.float32),
            pltpu.SemaphoreType.DMA,
            pltpu.SemaphoreType.DMA,
            pltpu.SemaphoreType.DMA,
            pltpu.SemaphoreType.DMA,
            pltpu.SemaphoreType.DMA,
            pltpu.SemaphoreType.DMA,
            pltpu.SemaphoreType.DMA,
            pltpu.SemaphoreType.DMA,
        ],
        compiler_params=pltpu.CompilerParams(use_tc_tiling_on_sc=False),
    )
    return f(stc, ftt, xpa, mfs)


def _k3_body(a0, a1, r_ref, b_ref, o_ref):
    asum = a0[...] + a1[...]
    n = asum[:, :HU]
    z = asum[:, HU:]
    zfull = jnp.dot(z, r_ref[...], preferred_element_type=jnp.float32,
                    precision=lax.Precision.HIGHEST) + 1e-7
    v = n / zfull + b_ref[...]
    o_ref[...] = jnp.where(v > 0, v, jnp.exp(jnp.minimum(v, 0.0)) - 1.0)


def _k3(a0, a1, r, bias2d):
    blk = 1000
    grid = N_NODES // blk
    return pl.pallas_call(
        _k3_body,
        grid=(grid,),
        in_specs=[
            pl.BlockSpec((blk, AW), lambda i: (i, 0)),
            pl.BlockSpec((blk, AW), lambda i: (i, 0)),
            pl.BlockSpec((16, HU), lambda i: (0, 0)),
            pl.BlockSpec((1, HU), lambda i: (0, 0)),
        ],
        out_specs=pl.BlockSpec((blk, HU), lambda i: (i, 0)),
        out_shape=jax.ShapeDtypeStruct((N_NODES, HU), jnp.float32),
    )(a0, a1, r, bias2d)


def kernel(x, edges, training, kernel, kernel_attention1, kernel_attention2,
           bias):
    del training
    sources = edges[:, 0].astype(jnp.int32)
    targets = edges[:, 1].astype(jnp.int32)
    pad = jnp.zeros((N_CHUNKS, TGO - CHUNK), jnp.int32)
    stc = jnp.concatenate(
        [sources.reshape(N_CHUNKS, CHUNK), pad,
         targets.reshape(N_CHUNKS, CHUNK), pad], axis=1)

    eye = jnp.eye(N_HEADS, dtype=jnp.float32)
    a1 = (kernel_attention1.reshape(N_HEADS, UNITS)[:, :, None]
          * eye[:, None, :]).reshape(HU, N_HEADS)
    a2 = (kernel_attention2.reshape(N_HEADS, UNITS)[:, :, None]
          * eye[:, None, :]).reshape(HU, N_HEADS)
    a_t = jnp.concatenate([a1, a1], axis=1)
    a_s = jnp.concatenate([a2, a2], axis=1)

    xpa, ftt, mfs2d = _k1(x, kernel, a_t, a_s)

    acc_p = _k2(stc, ftt, xpa, mfs2d.reshape(16))

    r = (jnp.arange(HU)[None, :] // UNITS
         == jnp.arange(16)[:, None]).astype(jnp.float32)
    out = _k3(acc_p[0], acc_p[1], r, bias.reshape(1, HU))
    return out


# speedup vs baseline: 1.0045x; 1.0045x over previous
import jax
import jax.numpy as jnp
from jax import lax
from jax.experimental import pallas as pl
from jax.experimental.pallas import tpu as pltpu
from jax.experimental.pallas import tpu_sc as plsc

N_NODES = 10000
N_EDGES = 320000
D_IN = 128
N_HEADS = 8
UNITS = 16
HU = N_HEADS * UNITS
AW = HU + 16

CHUNK = 100
N_CHUNKS = N_EDGES // CHUNK
NW = 32
NBASE = N_CHUNKS // NW
NQUAD = NBASE // 4
TGO = 112
IW = 2 * TGO
N_GROUPS = N_NODES // 16


def _k1_body(x_ref, w_ref, at_ref, as_ref, xpa_ref, ftt_ref, mf_ref):
    i = pl.program_id(0)
    xb = x_ref[...]
    xp = jnp.dot(xb, w_ref[...], preferred_element_type=jnp.float32,
                 precision=lax.Precision.HIGHEST)
    ftt_ref[...] = jnp.dot(xp, at_ref[...], preferred_element_type=jnp.float32,
                           precision=lax.Precision.HIGHEST)
    fss = jnp.dot(xp, as_ref[...], preferred_element_type=jnp.float32,
                  precision=lax.Precision.HIGHEST)
    xpa_ref[...] = jnp.concatenate([xp, fss], axis=1)
    bm = jnp.max(fss, axis=0, keepdims=True)

    @pl.when(i == 0)
    def _():
        mf_ref[...] = bm

    @pl.when(i > 0)
    def _():
        mf_ref[...] = jnp.maximum(mf_ref[...], bm)


def _k1(x, w, a_t, a_s):
    blk = 1000
    grid = N_NODES // blk
    return pl.pallas_call(
        _k1_body,
        grid=(grid,),
        in_specs=[
            pl.BlockSpec((blk, D_IN), lambda i: (i, 0)),
            pl.BlockSpec((D_IN, HU), lambda i: (0, 0)),
            pl.BlockSpec((D_IN, 16), lambda i: (0, 0)),
            pl.BlockSpec((D_IN, 16), lambda i: (0, 0)),
        ],
        out_specs=[
            pl.BlockSpec((blk, AW), lambda i: (i, 0)),
            pl.BlockSpec((blk, 16), lambda i: (i, 0)),
            pl.BlockSpec((1, 16), lambda i: (0, 0)),
        ],
        out_shape=[
            jax.ShapeDtypeStruct((N_NODES, AW), jnp.float32),
            jax.ShapeDtypeStruct((N_NODES, 16), jnp.float32),
            jax.ShapeDtypeStruct((1, 16), jnp.float32),
        ],
    )(x, w, a_t, a_s)


def _k2_body(stc_h, ftt_h, xpa_h, mfs_h,
             acc_o,
             ia0, ia1, ib0, ib1, xpbufa, xpbufb, tbufa, tbufb, tixa, tixb,
             mfs_v, nacc,
             sga, sgb, ssa, ssb, sia0, sia1, sib0, sib1):
    cid = lax.axis_index("c")
    sid = lax.axis_index("s")
    wid = sid * 2 + cid

    zer = jnp.zeros((16,), jnp.float32)

    def zb(k, c):
        xpbufa[k // 9, pl.ds((k % 9) * 16, 16)] = zer
        return c
    lax.fori_loop(0, 16 * 9, zb, 0)

    n_my_g = (N_GROUPS - sid + 15) // 16

    def zg(k, c):
        r0 = (sid + k * 16) * 16
        pltpu.sync_copy(xpbufa.at[pl.ds(0, 16)], nacc.at[pl.ds(r0, 16)])
        return c
    lax.fori_loop(0, n_my_g, zg, 0)
    plsc.subcore_barrier()

    pltpu.sync_copy(mfs_h, mfs_v)
    mfs = mfs_v[:]
    hidx = [jnp.full((16,), h, jnp.int32) for h in range(N_HEADS)]

    gdn = lax.GatherDimensionNumbers(
        offset_dims=(), collapsed_slice_dims=(0,), start_index_map=(0,))

    def take16(vec, idx):
        return lax.gather(
            vec, idx[:, None], gdn, (1,),
            mode=lax.GatherScatterMode.PROMISE_IN_BOUNDS)

    seta = (xpbufa, tbufa, tixa, sga, ssa)
    setb = (xpbufb, tbufb, tixb, sgb, ssb)

    def idx_dma(c, ibuf, isem):
        return pltpu.make_async_copy(stc_h.at[wid + c * NW], ibuf, isem)

    def g_descs(bufs, ibuf):
        xpbuf, tbuf, _, sg, _ = bufs
        return (
            pltpu.make_async_copy(xpa_h.at[ibuf.at[pl.ds(0, CHUNK)]],
                                  xpbuf, sg),
            pltpu.make_async_copy(ftt_h.at[ibuf.at[pl.ds(TGO, CHUNK)]],
                                  tbuf, sg),
        )

    def s_desc(bufs):
        xpbuf, _, tix, _, ss = bufs
        return pltpu.make_async_copy(xpbuf, nacc.at[tix], ss)

    def prefetch(bufs, ibuf, isem, first=False):
        _, _, tix, _, _ = bufs
        if not first:
            s_desc(bufs).wait()
        idx_dma(0, ibuf, isem).wait()
        for k in range(CHUNK // 16):
            tix[pl.ds(k * 16, 16)] = ibuf[pl.ds(TGO + k * 16, 16)]
        tix[pl.ds(CHUNK - 16, 16)] = ibuf[pl.ds(TGO + CHUNK - 16, 16)]
        for d in g_descs(bufs, ibuf):
            d.start()

    def process(bufs, ibuf, refill_c, isem):
        xpbuf, tbuf, tix, sg, ss = bufs
        for d in g_descs(bufs, ibuf):
            d.wait()

        @pl.when(refill_c < NBASE)
        def _():
            idx_dma(refill_c, ibuf, isem).start()

        @plsc.parallel_loop(0, CHUNK, unroll=2)
        def _(e):
            rt = tbuf[e, :]
            rs = xpbuf[e, pl.ds(HU, 16)]
            s = rt + rs
            s = jnp.maximum(s, 0.2 * s)
            cm = rt + mfs
            cm = jnp.maximum(cm, 0.2 * cm)
            p = jnp.exp(s - cm)
            xpbuf[e, pl.ds(HU, 16)] = p
            for h in range(N_HEADS):
                wv = take16(p, hidx[h])
                xv = xpbuf[e, pl.ds(h * UNITS, UNITS)]
                xpbuf[e, pl.ds(h * UNITS, UNITS)] = xv * wv

        pltpu.async_copy(xpbuf, nacc.at[tix], ss, add=True)

    idx_dma(0, ia0, sia0).start()
    idx_dma(1, ib0, sib0).start()
    idx_dma(2, ia1, sia1).start()
    idx_dma(3, ib1, sib1).start()
    prefetch(seta, ia0, sia0, first=True)
    prefetch(setb, ib0, sib0, first=True)

    def quad_body(k, cc):
        c = 4 * k
        process(seta, ia0, c + 4, sia0)
        prefetch(seta, ia1, sia1)
        process(setb, ib0, c + 5, sib0)
        prefetch(setb, ib1, sib1)
        process(seta, ia1, c + 6, sia1)

        @pl.when(c + 4 < NBASE)
        def _():
            prefetch(seta, ia0, sia0)
        process(setb, ib1, c + 7, sib1)

        @pl.when(c + 5 < NBASE)
        def _():
            prefetch(setb, ib0, sib0)
        return cc
    lax.fori_loop(0, NQUAD, quad_body, 0)

    s_desc(seta).wait()
    s_desc(setb).wait()

    plsc.subcore_barrier()

    def dg(k, c):
        r0 = (sid + k * 16) * 16
        pltpu.sync_copy(nacc.at[pl.ds(r0, 16)], acc_o.at[cid, pl.ds(r0, 16)])
        return c
    lax.fori_loop(0, n_my_g, dg, 0)


def _k2(stc, ftt, xpa, mfs):
    mesh = plsc.VectorSubcoreMesh(core_axis_name="c", subcore_axis_name="s")
    f = pl.kernel(
        _k2_body,
        mesh=mesh,
        out_type=jax.ShapeDtypeStruct((2, N_NODES, AW), jnp.float32),
        scratch_types=[
            pltpu.VMEM((IW,), jnp.int32),
            pltpu.VMEM((IW,), jnp.int32),
            pltpu.VMEM((IW,), jnp.int32),
            pltpu.VMEM((IW,), jnp.int32),
            pltpu.VMEM((CHUNK, AW), jnp.float32),
            pltpu.VMEM((CHUNK, AW), jnp.float32),
            pltpu.VMEM((CHUNK, 16), jnp.float32),
            pltpu.VMEM((CHUNK, 16), jnp.float32),
            pltpu.VMEM((CHUNK,), jnp.int32),
            pltpu.VMEM((CHUNK,), jnp.int32),
            pltpu.VMEM((16,), jnp.float32),
            pltpu.VMEM_SHARED((N_NODES, AW), jnp.float32),
            pltpu.SemaphoreType.DMA,
            pltpu.SemaphoreType.DMA,
            pltpu.SemaphoreType.DMA,
            pltpu.SemaphoreType.DMA,
            pltpu.SemaphoreType.DMA,
            pltpu.SemaphoreType.DMA,
            pltpu.SemaphoreType.DMA,
            pltpu.SemaphoreType.DMA,
        ],
        compiler_params=pltpu.CompilerParams(use_tc_tiling_on_sc=False),
    )
    return f(stc, ftt, xpa, mfs)


def _k3_body(a0, a1, r_ref, b_ref, o_ref):
    asum = a0[...] + a1[...]
    n = asum[:, :HU]
    z = asum[:, HU:]
    zfull = jnp.dot(z, r_ref[...], preferred_element_type=jnp.float32,
                    precision=lax.Precision.HIGHEST) + 1e-7
    v = n / zfull + b_ref[...]
    o_ref[...] = jnp.where(v > 0, v, jnp.exp(jnp.minimum(v, 0.0)) - 1.0)


def _k3(a0, a1, r, bias2d):
    blk = 1000
    grid = N_NODES // blk
    return pl.pallas_call(
        _k3_body,
        grid=(grid,),
        in_specs=[
            pl.BlockSpec((blk, AW), lambda i: (i, 0)),
            pl.BlockSpec((blk, AW), lambda i: (i, 0)),
            pl.BlockSpec((16, HU), lambda i: (0, 0)),
            pl.BlockSpec((1, HU), lambda i: (0, 0)),
        ],
        out_specs=pl.BlockSpec((blk, HU), lambda i: (i, 0)),
        out_shape=jax.ShapeDtypeStruct((N_NODES, HU), jnp.float32),
    )(a0, a1, r, bias2d)


def kernel(x, edges, training, kernel, kernel_attention1, kernel_attention2,
           bias):
    del training
    sources = edges[:, 0].astype(jnp.int32)
    targets = edges[:, 1].astype(jnp.int32)
    pad = jnp.zeros((N_CHUNKS, TGO - CHUNK), jnp.int32)
    stc = jnp.concatenate(
        [sources.reshape(N_CHUNKS, CHUNK), pad,
         targets.reshape(N_CHUNKS, CHUNK), pad], axis=1)

    eye = jnp.eye(N_HEADS, dtype=jnp.float32)
    a1 = (kernel_attention1.reshape(N_HEADS, UNITS)[:, :, None]
          * eye[:, None, :]).reshape(HU, N_HEADS)
    a2 = (kernel_attention2.reshape(N_HEADS, UNITS)[:, :, None]
          * eye[:, None, :]).reshape(HU, N_HEADS)
    a_t = jnp.concatenate([a1, a1], axis=1)
    a_s = jnp.concatenate([a2, a2], axis=1)

    xpa, ftt, mfs2d = _k1(x, kernel, a_t, a_s)

    acc_p = _k2(stc, ftt, xpa, mfs2d.reshape(16))

    r = (jnp.arange(HU)[None, :] // UNITS
         == jnp.arange(16)[:, None]).astype(jnp.float32)
    out = _k3(acc_p[0], acc_p[1], r, bias.reshape(1, HU))
    return out


# speedup vs baseline: 1.0178x; 1.0133x over previous
import jax
import jax.numpy as jnp
from jax import lax
from jax.experimental import pallas as pl
from jax.experimental.pallas import tpu as pltpu
from jax.experimental.pallas import tpu_sc as plsc

N_NODES = 10000
N_EDGES = 320000
D_IN = 128
N_HEADS = 8
UNITS = 16
HU = N_HEADS * UNITS
AW = HU + 16

CHUNK = 100
N_CHUNKS = N_EDGES // CHUNK
NW = 32
NBASE = N_CHUNKS // NW
NQUAD = NBASE // 4
TGO = 112
IW = 2 * TGO
HC = CHUNK // 2
N_GROUPS = N_NODES // 16


def _k1_body(x_ref, w_ref, at_ref, as_ref, xpa_ref, ftt_ref, mf_ref):
    i = pl.program_id(0)
    xb = x_ref[...]
    xp = jnp.dot(xb, w_ref[...], preferred_element_type=jnp.float32,
                 precision=lax.Precision.HIGHEST)
    ftt_ref[...] = jnp.dot(xp, at_ref[...], preferred_element_type=jnp.float32,
                           precision=lax.Precision.HIGHEST)
    fss = jnp.dot(xp, as_ref[...], preferred_element_type=jnp.float32,
                  precision=lax.Precision.HIGHEST)
    xpa_ref[...] = jnp.concatenate([xp, fss], axis=1)
    bm = jnp.max(fss, axis=0, keepdims=True)

    @pl.when(i == 0)
    def _():
        mf_ref[...] = bm

    @pl.when(i > 0)
    def _():
        mf_ref[...] = jnp.maximum(mf_ref[...], bm)


def _k1(x, w, a_t, a_s):
    blk = 1000
    grid = N_NODES // blk
    return pl.pallas_call(
        _k1_body,
        grid=(grid,),
        in_specs=[
            pl.BlockSpec((blk, D_IN), lambda i: (i, 0)),
            pl.BlockSpec((D_IN, HU), lambda i: (0, 0)),
            pl.BlockSpec((D_IN, 16), lambda i: (0, 0)),
            pl.BlockSpec((D_IN, 16), lambda i: (0, 0)),
        ],
        out_specs=[
            pl.BlockSpec((blk, AW), lambda i: (i, 0)),
            pl.BlockSpec((blk, 16), lambda i: (i, 0)),
            pl.BlockSpec((1, 16), lambda i: (0, 0)),
        ],
        out_shape=[
            jax.ShapeDtypeStruct((N_NODES, AW), jnp.float32),
            jax.ShapeDtypeStruct((N_NODES, 16), jnp.float32),
            jax.ShapeDtypeStruct((1, 16), jnp.float32),
        ],
    )(x, w, a_t, a_s)


def _k2_body(stc_h, ftt_h, xpa_h, mfs_h,
             acc_o,
             ia0, ia1, ib0, ib1, xpbufa, xpbufb, tbufa, tbufb,
             tixa0, tixa1, tixb0, tixb1,
             mfs_v, nacc,
             sga, sgb, ssa, ssb, sia0, sia1, sib0, sib1):
    cid = lax.axis_index("c")
    sid = lax.axis_index("s")
    wid = sid * 2 + cid

    zer = jnp.zeros((16,), jnp.float32)

    def zb(k, c):
        xpbufa[k // 9, pl.ds((k % 9) * 16, 16)] = zer
        return c
    lax.fori_loop(0, 16 * 9, zb, 0)

    n_my_g = (N_GROUPS - sid + 15) // 16

    def zg(k, c):
        r0 = (sid + k * 16) * 16
        pltpu.sync_copy(xpbufa.at[pl.ds(0, 16)], nacc.at[pl.ds(r0, 16)])
        return c
    lax.fori_loop(0, n_my_g, zg, 0)
    plsc.subcore_barrier()

    pltpu.sync_copy(mfs_h, mfs_v)
    mfs = mfs_v[:]
    hidx = [jnp.full((16,), h, jnp.int32) for h in range(N_HEADS)]

    gdn = lax.GatherDimensionNumbers(
        offset_dims=(), collapsed_slice_dims=(0,), start_index_map=(0,))

    def take16(vec, idx):
        return lax.gather(
            vec, idx[:, None], gdn, (1,),
            mode=lax.GatherScatterMode.PROMISE_IN_BOUNDS)

    seta = (xpbufa, tbufa, tixa0, tixa1, sga, ssa)
    setb = (xpbufb, tbufb, tixb0, tixb1, sgb, ssb)

    def idx_dma(c, ibuf, isem):
        return pltpu.make_async_copy(stc_h.at[wid + c * NW], ibuf, isem)

    def g_descs(bufs, ibuf):
        xpbuf, tbuf, _, _, sg, _ = bufs
        return (
            pltpu.make_async_copy(xpa_h.at[ibuf.at[pl.ds(0, CHUNK)]],
                                  xpbuf, sg),
            pltpu.make_async_copy(ftt_h.at[ibuf.at[pl.ds(TGO, CHUNK)]],
                                  tbuf, sg),
        )

    def s_descs(bufs):
        xpbuf, _, tix0, tix1, _, ss = bufs
        return (
            pltpu.make_async_copy(xpbuf.at[pl.ds(0, HC)], nacc.at[tix0], ss),
            pltpu.make_async_copy(xpbuf.at[pl.ds(HC, HC)], nacc.at[tix1], ss),
        )

    def prefetch(bufs, ibuf, isem, first=False):
        _, _, tix0, tix1, _, _ = bufs
        if not first:
            for d in s_descs(bufs):
                d.wait()
        idx_dma(0, ibuf, isem).wait()
        for k in range(3):
            tix0[pl.ds(k * 16, 16)] = ibuf[pl.ds(TGO + k * 16, 16)]
            tix1[pl.ds(k * 16, 16)] = ibuf[pl.ds(TGO + HC + k * 16, 16)]
        tix0[pl.ds(HC - 16, 16)] = ibuf[pl.ds(TGO + HC - 16, 16)]
        tix1[pl.ds(HC - 16, 16)] = ibuf[pl.ds(TGO + CHUNK - 16, 16)]
        for d in g_descs(bufs, ibuf):
            d.start()

    def process(bufs, ibuf, refill_c, isem):
        xpbuf, tbuf, tix0, tix1, sg, ss = bufs
        for d in g_descs(bufs, ibuf):
            d.wait()

        @pl.when(refill_c < NBASE)
        def _():
            idx_dma(refill_c, ibuf, isem).start()

        def edge_work(e):
            rt = tbuf[e, :]
            rs = xpbuf[e, pl.ds(HU, 16)]
            s = rt + rs
            s = jnp.maximum(s, 0.2 * s)
            cm = rt + mfs
            cm = jnp.maximum(cm, 0.2 * cm)
            p = jnp.exp(s - cm)
            xpbuf[e, pl.ds(HU, 16)] = p
            for h in range(N_HEADS):
                wv = take16(p, hidx[h])
                xv = xpbuf[e, pl.ds(h * UNITS, UNITS)]
                xpbuf[e, pl.ds(h * UNITS, UNITS)] = xv * wv

        plsc.parallel_loop(0, HC, unroll=2)(edge_work)
        pltpu.async_copy(xpbuf.at[pl.ds(0, HC)], nacc.at[tix0], ss, add=True)
        plsc.parallel_loop(HC, CHUNK, unroll=2)(edge_work)
        pltpu.async_copy(xpbuf.at[pl.ds(HC, HC)], nacc.at[tix1], ss, add=True)

    idx_dma(0, ia0, sia0).start()
    idx_dma(1, ib0, sib0).start()
    idx_dma(2, ia1, sia1).start()
    idx_dma(3, ib1, sib1).start()
    prefetch(seta, ia0, sia0, first=True)
    prefetch(setb, ib0, sib0, first=True)

    def quad_body(k, cc):
        c = 4 * k
        process(seta, ia0, c + 4, sia0)
        prefetch(seta, ia1, sia1)
        process(setb, ib0, c + 5, sib0)
        prefetch(setb, ib1, sib1)
        process(seta, ia1, c + 6, sia1)

        @pl.when(c + 4 < NBASE)
        def _():
            prefetch(seta, ia0, sia0)
        process(setb, ib1, c + 7, sib1)

        @pl.when(c + 5 < NBASE)
        def _():
            prefetch(setb, ib0, sib0)
        return cc
    lax.fori_loop(0, NQUAD, quad_body, 0)

    for d in s_descs(seta) + s_descs(setb):
        d.wait()

    plsc.subcore_barrier()

    def dg(k, c):
        r0 = (sid + k * 16) * 16
        pltpu.sync_copy(nacc.at[pl.ds(r0, 16)], acc_o.at[cid, pl.ds(r0, 16)])
        return c
    lax.fori_loop(0, n_my_g, dg, 0)


def _k2(stc, ftt, xpa, mfs):
    mesh = plsc.VectorSubcoreMesh(core_axis_name="c", subcore_axis_name="s")
    f = pl.kernel(
        _k2_body,
        mesh=mesh,
        out_type=jax.ShapeDtypeStruct((2, N_NODES, AW), jnp.float32),
        scratch_types=[
            pltpu.VMEM((IW,), jnp.int32),
            pltpu.VMEM((IW,), jnp.int32),
            pltpu.VMEM((IW,), jnp.int32),
            pltpu.VMEM((IW,), jnp.int32),
            pltpu.VMEM((CHUNK, AW), jnp.float32),
            pltpu.VMEM((CHUNK, AW), jnp.float32),
            pltpu.VMEM((CHUNK, 16), jnp.float32),
            pltpu.VMEM((CHUNK, 16), jnp.float32),
            pltpu.VMEM((HC,), jnp.int32),
            pltpu.VMEM((HC,), jnp.int32),
            pltpu.VMEM((HC,), jnp.int32),
            pltpu.VMEM((HC,), jnp.int32),
            pltpu.VMEM((16,), jnp.float32),
            pltpu.VMEM_SHARED((N_NODES, AW), jnp.float32),
            pltpu.SemaphoreType.DMA,
            pltpu.SemaphoreType.DMA,
            pltpu.SemaphoreType.DMA,
            pltpu.SemaphoreType.DMA,
            pltpu.SemaphoreType.DMA,
            pltpu.SemaphoreType.DMA,
            pltpu.SemaphoreType.DMA,
            pltpu.SemaphoreType.DMA,
        ],
        compiler_params=pltpu.CompilerParams(use_tc_tiling_on_sc=False),
    )
    return f(stc, ftt, xpa, mfs)


def _k3_body(a0, a1, r_ref, b_ref, o_ref):
    asum = a0[...] + a1[...]
    n = asum[:, :HU]
    z = asum[:, HU:]
    zfull = jnp.dot(z, r_ref[...], preferred_element_type=jnp.float32,
                    precision=lax.Precision.HIGHEST) + 1e-7
    v = n / zfull + b_ref[...]
    o_ref[...] = jnp.where(v > 0, v, jnp.exp(jnp.minimum(v, 0.0)) - 1.0)


def _k3(a0, a1, r, bias2d):
    blk = 1000
    grid = N_NODES // blk
    return pl.pallas_call(
        _k3_body,
        grid=(grid,),
        in_specs=[
            pl.BlockSpec((blk, AW), lambda i: (i, 0)),
            pl.BlockSpec((blk, AW), lambda i: (i, 0)),
            pl.BlockSpec((16, HU), lambda i: (0, 0)),
            pl.BlockSpec((1, HU), lambda i: (0, 0)),
        ],
        out_specs=pl.BlockSpec((blk, HU), lambda i: (i, 0)),
        out_shape=jax.ShapeDtypeStruct((N_NODES, HU), jnp.float32),
    )(a0, a1, r, bias2d)


def kernel(x, edges, training, kernel, kernel_attention1, kernel_attention2,
           bias):
    del training
    sources = edges[:, 0].astype(jnp.int32)
    targets = edges[:, 1].astype(jnp.int32)
    pad = jnp.zeros((N_CHUNKS, TGO - CHUNK), jnp.int32)
    stc = jnp.concatenate(
        [sources.reshape(N_CHUNKS, CHUNK), pad,
         targets.reshape(N_CHUNKS, CHUNK), pad], axis=1)

    eye = jnp.eye(N_HEADS, dtype=jnp.float32)
    a1 = (kernel_attention1.reshape(N_HEADS, UNITS)[:, :, None]
          * eye[:, None, :]).reshape(HU, N_HEADS)
    a2 = (kernel_attention2.reshape(N_HEADS, UNITS)[:, :, None]
          * eye[:, None, :]).reshape(HU, N_HEADS)
    a_t = jnp.concatenate([a1, a1], axis=1)
    a_s = jnp.concatenate([a2, a2], axis=1)

    xpa, ftt, mfs2d = _k1(x, kernel, a_t, a_s)

    acc_p = _k2(stc, ftt, xpa, mfs2d.reshape(16))

    r = (jnp.arange(HU)[None, :] // UNITS
         == jnp.arange(16)[:, None]).astype(jnp.float32)
    out = _k3(acc_p[0], acc_p[1], r, bias.reshape(1, HU))
    return out


# speedup vs baseline: 1.0704x; 1.0517x over previous
import jax
import jax.numpy as jnp
from jax import lax
from jax.experimental import pallas as pl
from jax.experimental.pallas import tpu as pltpu
from jax.experimental.pallas import tpu_sc as plsc

N_NODES = 10000
N_EDGES = 320000
D_IN = 128
N_HEADS = 8
UNITS = 16
HU = N_HEADS * UNITS
AW = HU + 16

CHUNK = 100
N_CHUNKS = N_EDGES // CHUNK
NW = 32
NBASE = N_CHUNKS // NW
NQUAD = NBASE // 4
TGO = 112
IW = 2 * TGO
HC = CHUNK // 2
N_GROUPS = N_NODES // 16


def _k1_body(x_ref, w_ref, at_ref, as_ref, xpa_ref, ftt_ref, mf_ref):
    i = pl.program_id(0)
    xb = x_ref[...]
    xp = jnp.dot(xb, w_ref[...], preferred_element_type=jnp.float32,
                 precision=lax.Precision.HIGHEST)
    ftt_ref[...] = jnp.dot(xp, at_ref[...], preferred_element_type=jnp.float32,
                           precision=lax.Precision.HIGHEST)
    fss = jnp.dot(xp, as_ref[...], preferred_element_type=jnp.float32,
                  precision=lax.Precision.HIGHEST)
    xpa_ref[...] = jnp.concatenate([xp, fss], axis=1)
    bm = jnp.max(fss, axis=0, keepdims=True)

    @pl.when(i == 0)
    def _():
        mf_ref[...] = bm

    @pl.when(i > 0)
    def _():
        mf_ref[...] = jnp.maximum(mf_ref[...], bm)


def _k1(x, w, a_t, a_s):
    blk = 2000
    grid = N_NODES // blk
    return pl.pallas_call(
        _k1_body,
        grid=(grid,),
        in_specs=[
            pl.BlockSpec((blk, D_IN), lambda i: (i, 0)),
            pl.BlockSpec((D_IN, HU), lambda i: (0, 0)),
            pl.BlockSpec((D_IN, 16), lambda i: (0, 0)),
            pl.BlockSpec((D_IN, 16), lambda i: (0, 0)),
        ],
        out_specs=[
            pl.BlockSpec((blk, AW), lambda i: (i, 0)),
            pl.BlockSpec((blk, 16), lambda i: (i, 0)),
            pl.BlockSpec((1, 16), lambda i: (0, 0)),
        ],
        out_shape=[
            jax.ShapeDtypeStruct((N_NODES, AW), jnp.float32),
            jax.ShapeDtypeStruct((N_NODES, 16), jnp.float32),
            jax.ShapeDtypeStruct((1, 16), jnp.float32),
        ],
    )(x, w, a_t, a_s)


def _k2_body(stc_h, ftt_h, xpa_h, mfs_h,
             acc_o,
             ia0, ia1, ib0, ib1, xpbufa, xpbufb, tbufa, tbufb,
             tixa0, tixa1, tixb0, tixb1,
             mfs_v, nacc,
             sga, sgb, ssa, ssb, sia0, sia1, sib0, sib1):
    cid = lax.axis_index("c")
    sid = lax.axis_index("s")
    wid = sid * 2 + cid

    zer = jnp.zeros((16,), jnp.float32)

    def zb(k, c):
        xpbufa[k // 9, pl.ds((k % 9) * 16, 16)] = zer
        return c
    lax.fori_loop(0, 16 * 9, zb, 0)

    n_my_g = (N_GROUPS - sid + 15) // 16

    def zg(k, c):
        r0 = (sid + k * 16) * 16
        pltpu.sync_copy(xpbufa.at[pl.ds(0, 16)], nacc.at[pl.ds(r0, 16)])
        return c
    lax.fori_loop(0, n_my_g, zg, 0)
    plsc.subcore_barrier()

    pltpu.sync_copy(mfs_h, mfs_v)
    mfs = mfs_v[:]
    hidx = [jnp.full((16,), h, jnp.int32) for h in range(N_HEADS)]

    gdn = lax.GatherDimensionNumbers(
        offset_dims=(), collapsed_slice_dims=(0,), start_index_map=(0,))

    def take16(vec, idx):
        return lax.gather(
            vec, idx[:, None], gdn, (1,),
            mode=lax.GatherScatterMode.PROMISE_IN_BOUNDS)

    seta = (xpbufa, tbufa, tixa0, tixa1, sga, ssa)
    setb = (xpbufb, tbufb, tixb0, tixb1, sgb, ssb)

    def idx_dma(c, ibuf, isem):
        return pltpu.make_async_copy(stc_h.at[wid + c * NW], ibuf, isem)

    def g_descs(bufs, ibuf):
        xpbuf, tbuf, _, _, sg, _ = bufs
        return (
            pltpu.make_async_copy(xpa_h.at[ibuf.at[pl.ds(0, CHUNK)]],
                                  xpbuf, sg),
            pltpu.make_async_copy(ftt_h.at[ibuf.at[pl.ds(TGO, CHUNK)]],
                                  tbuf, sg),
        )

    def s_descs(bufs):
        xpbuf, _, tix0, tix1, _, ss = bufs
        return (
            pltpu.make_async_copy(xpbuf.at[pl.ds(0, HC)], nacc.at[tix0], ss),
            pltpu.make_async_copy(xpbuf.at[pl.ds(HC, HC)], nacc.at[tix1], ss),
        )

    def prefetch(bufs, ibuf, isem, first=False):
        _, _, tix0, tix1, _, _ = bufs
        if not first:
            for d in s_descs(bufs):
                d.wait()
        idx_dma(0, ibuf, isem).wait()
        for k in range(3):
            tix0[pl.ds(k * 16, 16)] = ibuf[pl.ds(TGO + k * 16, 16)]
            tix1[pl.ds(k * 16, 16)] = ibuf[pl.ds(TGO + HC + k * 16, 16)]
        tix0[pl.ds(HC - 16, 16)] = ibuf[pl.ds(TGO + HC - 16, 16)]
        tix1[pl.ds(HC - 16, 16)] = ibuf[pl.ds(TGO + CHUNK - 16, 16)]
        for d in g_descs(bufs, ibuf):
            d.start()

    def process(bufs, ibuf, refill_c, isem):
        xpbuf, tbuf, tix0, tix1, sg, ss = bufs
        for d in g_descs(bufs, ibuf):
            d.wait()

        @pl.when(refill_c < NBASE)
        def _():
            idx_dma(refill_c, ibuf, isem).start()

        def edge_work(e):
            rt = tbuf[e, :]
            rs = xpbuf[e, pl.ds(HU, 16)]
            s = rt + rs
            s = jnp.maximum(s, 0.2 * s)
            cm = rt + mfs
            cm = jnp.maximum(cm, 0.2 * cm)
            p = jnp.exp(s - cm)
            xpbuf[e, pl.ds(HU, 16)] = p
            for h in range(N_HEADS):
                wv = take16(p, hidx[h])
                xv = xpbuf[e, pl.ds(h * UNITS, UNITS)]
                xpbuf[e, pl.ds(h * UNITS, UNITS)] = xv * wv

        plsc.parallel_loop(0, HC, unroll=2)(edge_work)
        pltpu.async_copy(xpbuf.at[pl.ds(0, HC)], nacc.at[tix0], ss, add=True)
        plsc.parallel_loop(HC, CHUNK, unroll=2)(edge_work)
        pltpu.async_copy(xpbuf.at[pl.ds(HC, HC)], nacc.at[tix1], ss, add=True)

    idx_dma(0, ia0, sia0).start()
    idx_dma(1, ib0, sib0).start()
    idx_dma(2, ia1, sia1).start()
    idx_dma(3, ib1, sib1).start()
    prefetch(seta, ia0, sia0, first=True)
    prefetch(setb, ib0, sib0, first=True)

    def quad_body(k, cc):
        c = 4 * k
        process(seta, ia0, c + 4, sia0)
        prefetch(seta, ia1, sia1)
        process(setb, ib0, c + 5, sib0)
        prefetch(setb, ib1, sib1)
        process(seta, ia1, c + 6, sia1)

        @pl.when(c + 4 < NBASE)
        def _():
            prefetch(seta, ia0, sia0)
        process(setb, ib1, c + 7, sib1)

        @pl.when(c + 5 < NBASE)
        def _():
            prefetch(setb, ib0, sib0)
        return cc
    lax.fori_loop(0, NQUAD, quad_body, 0)

    for d in s_descs(seta) + s_descs(setb):
        d.wait()

    plsc.subcore_barrier()

    def dg(k, c):
        r0 = (sid + k * 16) * 16
        pltpu.sync_copy(nacc.at[pl.ds(r0, 16)], acc_o.at[cid, pl.ds(r0, 16)])
        return c
    lax.fori_loop(0, n_my_g, dg, 0)


def _k2(stc, ftt, xpa, mfs):
    mesh = plsc.VectorSubcoreMesh(core_axis_name="c", subcore_axis_name="s")
    f = pl.kernel(
        _k2_body,
        mesh=mesh,
        out_type=jax.ShapeDtypeStruct((2, N_NODES, AW), jnp.float32),
        scratch_types=[
            pltpu.VMEM((IW,), jnp.int32),
            pltpu.VMEM((IW,), jnp.int32),
            pltpu.VMEM((IW,), jnp.int32),
            pltpu.VMEM((IW,), jnp.int32),
            pltpu.VMEM((CHUNK, AW), jnp.float32),
            pltpu.VMEM((CHUNK, AW), jnp.float32),
            pltpu.VMEM((CHUNK, 16), jnp.float32),
            pltpu.VMEM((CHUNK, 16), jnp.float32),
            pltpu.VMEM((HC,), jnp.int32),
            pltpu.VMEM((HC,), jnp.int32),
            pltpu.VMEM((HC,), jnp.int32),
            pltpu.VMEM((HC,), jnp.int32),
            pltpu.VMEM((16,), jnp.float32),
            pltpu.VMEM_SHARED((N_NODES, AW), jnp.float32),
            pltpu.SemaphoreType.DMA,
            pltpu.SemaphoreType.DMA,
            pltpu.SemaphoreType.DMA,
            pltpu.SemaphoreType.DMA,
            pltpu.SemaphoreType.DMA,
            pltpu.SemaphoreType.DMA,
            pltpu.SemaphoreType.DMA,
            pltpu.SemaphoreType.DMA,
        ],
        compiler_params=pltpu.CompilerParams(use_tc_tiling_on_sc=False),
    )
    return f(stc, ftt, xpa, mfs)


def _k3_body(a0, a1, r_ref, b_ref, o_ref):
    asum = a0[...] + a1[...]
    n = asum[:, :HU]
    z = asum[:, HU:]
    zfull = jnp.dot(z, r_ref[...], preferred_element_type=jnp.float32,
                    precision=lax.Precision.HIGHEST) + 1e-7
    v = n / zfull + b_ref[...]
    o_ref[...] = jnp.where(v > 0, v, jnp.exp(jnp.minimum(v, 0.0)) - 1.0)


def _k3(a0, a1, r, bias2d):
    blk = 2000
    grid = N_NODES // blk
    return pl.pallas_call(
        _k3_body,
        grid=(grid,),
        in_specs=[
            pl.BlockSpec((blk, AW), lambda i: (i, 0)),
            pl.BlockSpec((blk, AW), lambda i: (i, 0)),
            pl.BlockSpec((16, HU), lambda i: (0, 0)),
            pl.BlockSpec((1, HU), lambda i: (0, 0)),
        ],
        out_specs=pl.BlockSpec((blk, HU), lambda i: (i, 0)),
        out_shape=jax.ShapeDtypeStruct((N_NODES, HU), jnp.float32),
    )(a0, a1, r, bias2d)


def kernel(x, edges, training, kernel, kernel_attention1, kernel_attention2,
           bias):
    del training
    sources = edges[:, 0].astype(jnp.int32)
    targets = edges[:, 1].astype(jnp.int32)
    pad = jnp.zeros((N_CHUNKS, TGO - CHUNK), jnp.int32)
    stc = jnp.concatenate(
        [sources.reshape(N_CHUNKS, CHUNK), pad,
         targets.reshape(N_CHUNKS, CHUNK), pad], axis=1)

    eye = jnp.eye(N_HEADS, dtype=jnp.float32)
    a1 = (kernel_attention1.reshape(N_HEADS, UNITS)[:, :, None]
          * eye[:, None, :]).reshape(HU, N_HEADS)
    a2 = (kernel_attention2.reshape(N_HEADS, UNITS)[:, :, None]
          * eye[:, None, :]).reshape(HU, N_HEADS)
    a_t = jnp.concatenate([a1, a1], axis=1)
    a_s = jnp.concatenate([a2, a2], axis=1)

    xpa, ftt, mfs2d = _k1(x, kernel, a_t, a_s)

    acc_p = _k2(stc, ftt, xpa, mfs2d.reshape(16))

    r = (jnp.arange(HU)[None, :] // UNITS
         == jnp.arange(16)[:, None]).astype(jnp.float32)
    out = _k3(acc_p[0], acc_p[1], r, bias.reshape(1, HU))
    return out
